# Initial kernel scaffold; baseline (speedup 1.0000x reference)
import jax
import jax.numpy as jnp
from jax.experimental import pallas as pl


def kernel(gene_embedded, value, shared_W, routing_W, router_W1, router_W2):
    raise NotImplementedError("write your pallas kernel here")


# speedup vs baseline: 18.5187x; 18.5187x over previous
import jax
import jax.numpy as jnp
from jax.experimental import pallas as pl

B, T, D = 4, 2048, 1024
NS, NR, K = 2, 8, 2
M_BLK = 512


def _fused_kernel(x_ref, v_ref, sw_ref, rw_ref, w1_ref, w2_ref, out_ref):
    x = x_ref[...]
    h = jax.lax.dot_general(
        x, w1_ref[...],
        dimension_numbers=(((1,), (1,)), ((), ())),
        preferred_element_type=jnp.float32,
    )
    h = jnp.maximum(h, 0.0)
    logits = jax.lax.dot_general(
        h, w2_ref[...],
        dimension_numbers=(((1,), (1,)), ((), ())),
        preferred_element_type=jnp.float32,
    )
    m = jnp.max(logits, axis=-1, keepdims=True)
    e = jnp.exp(logits - m)
    w = e / jnp.sum(e, axis=-1, keepdims=True)

    cols = jax.lax.broadcasted_iota(jnp.int32, w.shape, 1)
    m1 = jnp.max(w, axis=-1, keepdims=True)
    c1 = jnp.min(jnp.where(w == m1, cols, NR), axis=-1, keepdims=True)
    mask1 = cols == c1
    w_rest = jnp.where(mask1, -jnp.inf, w)
    m2 = jnp.max(w_rest, axis=-1, keepdims=True)
    c2 = jnp.min(jnp.where(w_rest == m2, cols, NR), axis=-1, keepdims=True)
    sw = jnp.where(mask1 | (cols == c2), w, 0.0)

    comb = jax.lax.dot_general(
        sw, rw_ref[...],
        dimension_numbers=(((1,), (0,)), ((), ())),
        preferred_element_type=jnp.float32,
    )
    wsum = jnp.sum(sw_ref[...], axis=0, keepdims=True)
    v = v_ref[...].reshape(-1, 1)
    out_ref[...] = v * (wsum + comb)


def kernel(gene_embedded, value, shared_W, routing_W, router_W1, router_W2):
    N = B * T
    x = gene_embedded.reshape(N, D)
    v = value.reshape(N)
    grid = N // M_BLK
    out = pl.pallas_call(
        _fused_kernel,
        grid=(grid,),
        in_specs=[
            pl.BlockSpec((M_BLK, D), lambda i: (i, 0)),
            pl.BlockSpec((M_BLK,), lambda i: (i,)),
            pl.BlockSpec((NS, D), lambda i: (0, 0)),
            pl.BlockSpec((NR, D), lambda i: (0, 0)),
            pl.BlockSpec((D, D), lambda i: (0, 0)),
            pl.BlockSpec((NR, D), lambda i: (0, 0)),
        ],
        out_specs=pl.BlockSpec((M_BLK, D), lambda i: (i, 0)),
        out_shape=jax.ShapeDtypeStruct((N, D), jnp.float32),
    )(x, v, shared_W, routing_W, router_W1, router_W2)
    return out.reshape(B, T, D)


# speedup vs baseline: 18.5252x; 1.0004x over previous
import jax
import jax.numpy as jnp
from jax.experimental import pallas as pl

B, T, D = 4, 2048, 1024
NS, NR, K = 2, 8, 2
M_BLK = 512


def _fused_kernel(x_ref, v_ref, sw_ref, rw_ref, w1_ref, w2_ref, out_ref):
    x = x_ref[...].astype(jnp.bfloat16)
    h = jax.lax.dot_general(
        x, w1_ref[...].astype(jnp.bfloat16),
        dimension_numbers=(((1,), (1,)), ((), ())),
        preferred_element_type=jnp.float32,
    )
    h = jnp.maximum(h, 0.0)
    logits = jax.lax.dot_general(
        h, w2_ref[...],
        dimension_numbers=(((1,), (1,)), ((), ())),
        preferred_element_type=jnp.float32,
    )
    m = jnp.max(logits, axis=-1, keepdims=True)
    e = jnp.exp(logits - m)
    w = e / jnp.sum(e, axis=-1, keepdims=True)

    cols = jax.lax.broadcasted_iota(jnp.int32, w.shape, 1)
    m1 = jnp.max(w, axis=-1, keepdims=True)
    c1 = jnp.min(jnp.where(w == m1, cols, NR), axis=-1, keepdims=True)
    mask1 = cols == c1
    w_rest = jnp.where(mask1, -jnp.inf, w)
    m2 = jnp.max(w_rest, axis=-1, keepdims=True)
    c2 = jnp.min(jnp.where(w_rest == m2, cols, NR), axis=-1, keepdims=True)
    sw = jnp.where(mask1 | (cols == c2), w, 0.0)

    comb = jax.lax.dot_general(
        sw, rw_ref[...],
        dimension_numbers=(((1,), (0,)), ((), ())),
        preferred_element_type=jnp.float32,
    )
    wsum = jnp.sum(sw_ref[...], axis=0, keepdims=True)
    v = v_ref[...].reshape(-1, 1)
    out_ref[...] = v * (wsum + comb)


def kernel(gene_embedded, value, shared_W, routing_W, router_W1, router_W2):
    N = B * T
    x = gene_embedded.reshape(N, D)
    v = value.reshape(N)
    grid = N // M_BLK
    out = pl.pallas_call(
        _fused_kernel,
        grid=(grid,),
        in_specs=[
            pl.BlockSpec((M_BLK, D), lambda i: (i, 0)),
            pl.BlockSpec((M_BLK,), lambda i: (i,)),
            pl.BlockSpec((NS, D), lambda i: (0, 0)),
            pl.BlockSpec((NR, D), lambda i: (0, 0)),
            pl.BlockSpec((D, D), lambda i: (0, 0)),
            pl.BlockSpec((NR, D), lambda i: (0, 0)),
        ],
        out_specs=pl.BlockSpec((M_BLK, D), lambda i: (i, 0)),
        out_shape=jax.ShapeDtypeStruct((N, D), jnp.float32),
    )(x, v, shared_W, routing_W, router_W1, router_W2)
    return out.reshape(B, T, D)


# speedup vs baseline: 25.0682x; 1.3532x over previous
import jax
import jax.numpy as jnp
from jax.experimental import pallas as pl

B, T, D = 4, 2048, 1024
NS, NR, K = 2, 8, 2
M_BLK = 512


def _fused_kernel(x_ref, v_ref, sw_ref, rw_ref, w1_ref, w2_ref, out_ref):
    x = x_ref[...].astype(jnp.bfloat16)
    h = jax.lax.dot_general(
        x, w1_ref[...].astype(jnp.bfloat16),
        dimension_numbers=(((1,), (1,)), ((), ())),
        preferred_element_type=jnp.float32,
    )
    h = jnp.maximum(h, 0.0)
    logits = jax.lax.dot_general(
        w2_ref[...], h,
        dimension_numbers=(((1,), (1,)), ((), ())),
        preferred_element_type=jnp.float32,
    )
    m = jnp.max(logits, axis=0, keepdims=True)
    e = jnp.exp(logits - m)
    w = e / jnp.sum(e, axis=0, keepdims=True)

    rows = jax.lax.broadcasted_iota(jnp.int32, w.shape, 0)
    m1 = jnp.max(w, axis=0, keepdims=True)
    c1 = jnp.min(jnp.where(w == m1, rows, NR), axis=0, keepdims=True)
    mask1 = rows == c1
    w_rest = jnp.where(mask1, -jnp.inf, w)
    m2 = jnp.max(w_rest, axis=0, keepdims=True)
    c2 = jnp.min(jnp.where(w_rest == m2, rows, NR), axis=0, keepdims=True)
    sw = jnp.where(mask1 | (rows == c2), w, 0.0)

    comb = jax.lax.dot_general(
        sw, rw_ref[...],
        dimension_numbers=(((0,), (0,)), ((), ())),
        preferred_element_type=jnp.float32,
    )
    wsum = jnp.sum(sw_ref[...], axis=0, keepdims=True)
    v = v_ref[...].reshape(-1, 1)
    out_ref[...] = v * (wsum + comb)


def kernel(gene_embedded, value, shared_W, routing_W, router_W1, router_W2):
    N = B * T
    x = gene_embedded.reshape(N, D)
    v = value.reshape(N)
    grid = N // M_BLK
    out = pl.pallas_call(
        _fused_kernel,
        grid=(grid,),
        in_specs=[
            pl.BlockSpec((M_BLK, D), lambda i: (i, 0)),
            pl.BlockSpec((M_BLK,), lambda i: (i,)),
            pl.BlockSpec((NS, D), lambda i: (0, 0)),
            pl.BlockSpec((NR, D), lambda i: (0, 0)),
            pl.BlockSpec((D, D), lambda i: (0, 0)),
            pl.BlockSpec((NR, D), lambda i: (0, 0)),
        ],
        out_specs=pl.BlockSpec((M_BLK, D), lambda i: (i, 0)),
        out_shape=jax.ShapeDtypeStruct((N, D), jnp.float32),
    )(x, v, shared_W, routing_W, router_W1, router_W2)
    return out.reshape(B, T, D)
